# baseline (speedup 1.0000x reference)
import functools

import jax
import jax.numpy as jnp
from jax import lax
from jax.experimental import pallas as pl
from jax.experimental.pallas import tpu as pltpu
from jax.experimental.pallas import tpu_sc as plsc

N = 10000
E = 320000
D = 128
G = 64
T = 4

NC = 2
NS = 16
NW = NC * NS
EPW = E // NW
K = 80
CPT = EPW // K
NPAD = 10240
RPT = NPAD // NS
SROWS = RPT // 5

_mesh = plsc.VectorSubcoreMesh(core_axis_name="c", subcore_axis_name="s")


@functools.partial(
    pl.kernel,
    out_type=jax.ShapeDtypeStruct((E, D), jnp.float32),
    mesh=_mesh,
    scratch_types=[
        pltpu.VMEM((EPW,), jnp.int32),
        pltpu.VMEM((EPW,), jnp.int32),
        pltpu.VMEM((2, K, D), jnp.float32),
        pltpu.VMEM((2, K, D), jnp.float32),
        pltpu.SemaphoreType.DMA,
        pltpu.SemaphoreType.DMA,
    ],
)
def _sc_gather(a_hbm, b_hbm, first_hbm, second_hbm, c_hbm,
               idx_a, idx_b, buf_a, buf_b, sem_g, sem_w):
    wid = lax.axis_index("s") * NC + lax.axis_index("c")
    e0 = wid * EPW

    pltpu.sync_copy(first_hbm.at[pl.ds(e0, EPW)], idx_a)
    pltpu.sync_copy(second_hbm.at[pl.ds(e0, EPW)], idx_b)

    def start_gather(ci, slot):
        pltpu.async_copy(a_hbm.at[idx_a.at[pl.ds(ci * K, K)]], buf_a.at[slot], sem_g)
        pltpu.async_copy(b_hbm.at[idx_b.at[pl.ds(ci * K, K)]], buf_b.at[slot], sem_g)

    def wait_gather(slot):
        pltpu.make_async_copy(a_hbm.at[idx_a.at[pl.ds(0, K)]], buf_a.at[slot], sem_g).wait()
        pltpu.make_async_copy(b_hbm.at[idx_b.at[pl.ds(0, K)]], buf_b.at[slot], sem_g).wait()

    def start_write(ci, slot):
        pltpu.async_copy(buf_a.at[slot], c_hbm.at[pl.ds(e0 + ci * K, K)], sem_w)

    def wait_write(slot):
        pltpu.make_async_copy(buf_a.at[slot], c_hbm.at[pl.ds(0, K)], sem_w).wait()

    start_gather(0, 0)

    def body(ci, carry):
        slot = lax.rem(ci, 2)
        nslot = 1 - slot

        @pl.when(ci + 1 < CPT)
        def _():
            @pl.when(ci >= 1)
            def _():
                wait_write(nslot)
            start_gather(ci + 1, nslot)

        wait_gather(slot)

        def ew(r, carry2):
            for j in range(D // 16):
                v = buf_a[slot, r, pl.ds(j * 16, 16)] + buf_b[slot, r, pl.ds(j * 16, 16)]
                buf_a[slot, r, pl.ds(j * 16, 16)] = jnp.maximum(v, 0.0)
            return carry2

        lax.fori_loop(0, K, ew, 0)
        start_write(ci, slot)
        return carry

    lax.fori_loop(0, CPT, body, None)
    wait_write(0)
    wait_write(1)


@functools.partial(
    pl.kernel,
    out_type=jax.ShapeDtypeStruct((NC, NPAD, D), jnp.float32),
    mesh=_mesh,
    scratch_types=[
        pltpu.VMEM((2, K), jnp.int32),
        pltpu.VMEM((2, K, D), jnp.float32),
        pltpu.VMEM((SROWS, D), jnp.float32),
        pltpu.VMEM_SHARED((NPAD, D), jnp.float32),
        pltpu.SemaphoreType.DMA,
    ],
)
def _sc_scatter(m_hbm, second_hbm, zero_hbm, out_hbm,
                idx_s, mbuf, stage, acc, sem_g):
    cid = lax.axis_index("c")
    sid = lax.axis_index("s")
    wid = sid * NC + cid
    e0 = wid * EPW

    pltpu.sync_copy(zero_hbm, stage)
    for k in range(5):
        pltpu.sync_copy(stage, acc.at[pl.ds(sid * RPT + k * SROWS, SROWS)])
    plsc.subcore_barrier()

    def start_load(ci, slot):
        base = e0 + ci * K
        pltpu.async_copy(second_hbm.at[pl.ds(base, K)], idx_s.at[slot], sem_g)
        pltpu.async_copy(m_hbm.at[pl.ds(base, K)], mbuf.at[slot], sem_g)

    def wait_load(slot):
        pltpu.make_async_copy(second_hbm.at[pl.ds(0, K)], idx_s.at[slot], sem_g).wait()
        pltpu.make_async_copy(m_hbm.at[pl.ds(0, K)], mbuf.at[slot], sem_g).wait()

    start_load(0, 0)

    def body(ci, carry):
        slot = lax.rem(ci, 2)

        @pl.when(ci + 1 < CPT)
        def _():
            start_load(ci + 1, 1 - slot)

        wait_load(slot)
        pltpu.sync_copy(mbuf.at[slot], acc.at[idx_s.at[slot]], add=True)
        return carry

    lax.fori_loop(0, CPT, body, None)
    plsc.subcore_barrier()

    for k in range(5):
        r0 = sid * RPT + k * SROWS
        pltpu.sync_copy(acc.at[pl.ds(r0, SROWS)], stage)
        pltpu.sync_copy(stage, out_hbm.at[cid, pl.ds(r0, SROWS)])


BN = 2000
BE = 4000


def _proj_body(h_ref, w1a_ref, w1b_ref, b1_ref, a_ref, b_ref):
    h = h_ref[...]
    a_ref[...] = jnp.dot(h, w1a_ref[...], preferred_element_type=jnp.float32) + b1_ref[...]
    b_ref[...] = jnp.dot(h, w1b_ref[...], preferred_element_type=jnp.float32)


_proj = pl.pallas_call(
    _proj_body,
    grid=(N // BN,),
    in_specs=[
        pl.BlockSpec((BN, D), lambda i: (i, 0)),
        pl.BlockSpec((D, D), lambda i: (0, 0)),
        pl.BlockSpec((D, D), lambda i: (0, 0)),
        pl.BlockSpec((1, D), lambda i: (0, 0)),
    ],
    out_specs=[
        pl.BlockSpec((BN, D), lambda i: (i, 0)),
        pl.BlockSpec((BN, D), lambda i: (i, 0)),
    ],
    out_shape=[
        jax.ShapeDtypeStruct((N, D), jnp.float32),
        jax.ShapeDtypeStruct((N, D), jnp.float32),
    ],
)


def _edge_mlp_body(c_ref, w2_ref, b2_ref, m_ref):
    m = jnp.dot(c_ref[...], w2_ref[...], preferred_element_type=jnp.float32) + b2_ref[...]
    m_ref[...] = jnp.maximum(m, 0.0)


_edge_mlp = pl.pallas_call(
    _edge_mlp_body,
    grid=(E // BE,),
    in_specs=[
        pl.BlockSpec((BE, D), lambda i: (i, 0)),
        pl.BlockSpec((D, D), lambda i: (0, 0)),
        pl.BlockSpec((1, D), lambda i: (0, 0)),
    ],
    out_specs=pl.BlockSpec((BE, D), lambda i: (i, 0)),
    out_shape=jax.ShapeDtypeStruct((E, D), jnp.float32),
)


def _gru_proj_body(p_ref, h_ref, gk_ref, grk_ref, gb_ref,
                   w1a_ref, w1b_ref, b1_ref, ho_ref, a_ref, b_ref):
    x = p_ref[0] + p_ref[1]
    h = h_ref[...]
    mx = jnp.dot(x, gk_ref[...], preferred_element_type=jnp.float32) + gb_ref[0:1, :]
    mh = jnp.dot(h, grk_ref[...], preferred_element_type=jnp.float32) + gb_ref[1:2, :]
    z = jax.nn.sigmoid(mx[:, :D] + mh[:, :D])
    r = jax.nn.sigmoid(mx[:, D:2 * D] + mh[:, D:2 * D])
    hh = jnp.tanh(mx[:, 2 * D:] + r * mh[:, 2 * D:])
    hn = z * h + (1.0 - z) * hh
    ho_ref[...] = hn
    a_ref[...] = jnp.dot(hn, w1a_ref[...], preferred_element_type=jnp.float32) + b1_ref[...]
    b_ref[...] = jnp.dot(hn, w1b_ref[...], preferred_element_type=jnp.float32)


_gru_proj = pl.pallas_call(
    _gru_proj_body,
    grid=(N // BN,),
    in_specs=[
        pl.BlockSpec((NC, BN, D), lambda i: (0, i, 0)),
        pl.BlockSpec((BN, D), lambda i: (i, 0)),
        pl.BlockSpec((D, 3 * D), lambda i: (0, 0)),
        pl.BlockSpec((D, 3 * D), lambda i: (0, 0)),
        pl.BlockSpec((2, 3 * D), lambda i: (0, 0)),
        pl.BlockSpec((D, D), lambda i: (0, 0)),
        pl.BlockSpec((D, D), lambda i: (0, 0)),
        pl.BlockSpec((1, D), lambda i: (0, 0)),
    ],
    out_specs=[
        pl.BlockSpec((BN, D), lambda i: (i, 0)),
        pl.BlockSpec((BN, D), lambda i: (i, 0)),
        pl.BlockSpec((BN, D), lambda i: (i, 0)),
    ],
    out_shape=[
        jax.ShapeDtypeStruct((N, D), jnp.float32),
        jax.ShapeDtypeStruct((N, D), jnp.float32),
        jax.ShapeDtypeStruct((N, D), jnp.float32),
    ],
)


def _readout_body(ids_ref, h_ref, wr1_ref, br1_ref, wr2_ref, br2_ref,
                  mask_ref, out_ref, acc_ref):
    i = pl.program_id(0)

    @pl.when(i == 0)
    def _zero():
        acc_ref[...] = jnp.zeros_like(acc_ref)

    ids = ids_ref[0]
    seg = lax.broadcasted_iota(jnp.int32, (G, 1), 0)
    onehot = (ids == seg).astype(jnp.float32)
    acc_ref[...] += jnp.dot(onehot, h_ref[...], preferred_element_type=jnp.float32)

    @pl.when(i == N // BN - 1)
    def _finish():
        t = jnp.dot(acc_ref[...], wr1_ref[...], preferred_element_type=jnp.float32)
        t = jnp.maximum(t + br1_ref[...], 0.0) * mask_ref[...]
        out_ref[...] = jnp.sum(t * wr2_ref[...], axis=1, keepdims=True) + br2_ref[...]


_readout = pl.pallas_call(
    _readout_body,
    grid=(N // BN,),
    in_specs=[
        pl.BlockSpec((1, 1, BN), lambda i: (i, 0, 0)),
        pl.BlockSpec((BN, D), lambda i: (i, 0)),
        pl.BlockSpec((D, D), lambda i: (0, 0)),
        pl.BlockSpec((1, D), lambda i: (0, 0)),
        pl.BlockSpec((1, D), lambda i: (0, 0)),
        pl.BlockSpec((1, 1), lambda i: (0, 0)),
        pl.BlockSpec((1, D), lambda i: (0, 0)),
    ],
    out_specs=pl.BlockSpec((G, 1), lambda i: (0, 0)),
    out_shape=jax.ShapeDtypeStruct((G, 1), jnp.float32),
    scratch_shapes=[pltpu.VMEM((G, D), jnp.float32)],
)


def kernel(link_state, states_graph_ids, states_first, states_second,
           sates_num_edges, Wm1, bm1, Wm2, bm2, gru_kernel, gru_rkernel,
           gru_bias, Wr1, br1, Wr2, br2, mask0):
    w1a = Wm1[:D]
    w1b = Wm1[D:]
    b1 = bm1.reshape(1, D)
    b2 = bm2.reshape(1, D)
    zeros_stage = jnp.zeros((SROWS, D), jnp.float32)

    h = link_state
    a, b = _proj(h, w1a, w1b, b1)
    for _ in range(T):
        c = _sc_gather(a, b, states_first, states_second)
        m = _edge_mlp(c, Wm2, b2)
        p = _sc_scatter(m, states_second, zeros_stage)
        h, a, b = _gru_proj(p, h, gru_kernel, gru_rkernel, gru_bias,
                            w1a, w1b, b1)

    out = _readout(states_graph_ids.reshape(N // BN, 1, BN), h, Wr1,
                   br1.reshape(1, D), Wr2.reshape(1, D), br2.reshape(1, 1),
                   mask0.reshape(1, D))
    return out


# speedup vs baseline: 1.3987x; 1.3987x over previous
import functools

import jax
import jax.numpy as jnp
from jax import lax
from jax.experimental import pallas as pl
from jax.experimental.pallas import tpu as pltpu
from jax.experimental.pallas import tpu_sc as plsc

N = 10000
E = 320000
D = 128
G = 64
T = 4

NC = 2
NS = 16
NW = NC * NS
EH = E // 2
NPAD = 10240
RPT = NPAD // NS
SROWS = RPT // 5

_mesh = plsc.VectorSubcoreMesh(core_axis_name="c", subcore_axis_name="s")


def _make_gather(ne, k):
    epw = ne // NW
    cpt = epw // k

    @functools.partial(
        pl.kernel,
        out_type=(
            jax.ShapeDtypeStruct((ne, D), jnp.float32),
            jax.ShapeDtypeStruct((ne, D), jnp.float32),
        ),
        mesh=_mesh,
        scratch_types=[
            pltpu.VMEM((epw,), jnp.int32),
            pltpu.VMEM((epw,), jnp.int32),
            pltpu.VMEM((2, k, D), jnp.float32),
            pltpu.VMEM((2, k, D), jnp.float32),
            pltpu.SemaphoreType.DMA,
            pltpu.SemaphoreType.DMA,
        ],
    )
    def gather(a_hbm, b_hbm, first_hbm, second_hbm, c1_hbm, c2_hbm,
               idx_a, idx_b, buf_a, buf_b, sem_g, sem_w):
        wid = lax.axis_index("s") * NC + lax.axis_index("c")
        e0 = wid * epw

        pltpu.sync_copy(first_hbm.at[pl.ds(e0, epw)], idx_a)
        pltpu.sync_copy(second_hbm.at[pl.ds(e0, epw)], idx_b)

        def start_gather(ci, slot):
            pltpu.async_copy(a_hbm.at[idx_a.at[pl.ds(ci * k, k)]], buf_a.at[slot], sem_g)
            pltpu.async_copy(b_hbm.at[idx_b.at[pl.ds(ci * k, k)]], buf_b.at[slot], sem_g)

        def wait_gather(slot):
            pltpu.make_async_copy(a_hbm.at[idx_a.at[pl.ds(0, k)]], buf_a.at[slot], sem_g).wait()
            pltpu.make_async_copy(b_hbm.at[idx_b.at[pl.ds(0, k)]], buf_b.at[slot], sem_g).wait()

        def start_write(ci, slot):
            base = e0 + ci * k
            pltpu.async_copy(buf_a.at[slot], c1_hbm.at[pl.ds(base, k)], sem_w)
            pltpu.async_copy(buf_b.at[slot], c2_hbm.at[pl.ds(base, k)], sem_w)

        def wait_write(slot):
            pltpu.make_async_copy(buf_a.at[slot], c1_hbm.at[pl.ds(0, k)], sem_w).wait()
            pltpu.make_async_copy(buf_b.at[slot], c2_hbm.at[pl.ds(0, k)], sem_w).wait()

        start_gather(0, 0)

        def body(ci, carry):
            slot = lax.rem(ci, 2)
            nslot = 1 - slot

            @pl.when(ci + 1 < cpt)
            def _():
                @pl.when(ci >= 1)
                def _():
                    wait_write(nslot)
                start_gather(ci + 1, nslot)

            wait_gather(slot)
            start_write(ci, slot)
            return carry

        lax.fori_loop(0, cpt, body, None)
        wait_write(0)
        wait_write(1)

    return gather


def _make_scatter(ne, k):
    epw = ne // NW
    cpt = epw // k

    @functools.partial(
        pl.kernel,
        out_type=jax.ShapeDtypeStruct((NC, NPAD, D), jnp.float32),
        mesh=_mesh,
        scratch_types=[
            pltpu.VMEM((2, k), jnp.int32),
            pltpu.VMEM((2, k, D), jnp.float32),
            pltpu.VMEM((SROWS, D), jnp.float32),
            pltpu.VMEM_SHARED((NPAD, D), jnp.float32),
            pltpu.SemaphoreType.DMA,
        ],
    )
    def scatter(m_hbm, second_hbm, zero_hbm, out_hbm,
                idx_s, mbuf, stage, acc, sem_g):
        cid = lax.axis_index("c")
        sid = lax.axis_index("s")
        wid = sid * NC + cid
        e0 = wid * epw

        pltpu.sync_copy(zero_hbm, stage)
        for kk in range(5):
            pltpu.sync_copy(stage, acc.at[pl.ds(sid * RPT + kk * SROWS, SROWS)])
        plsc.subcore_barrier()

        def start_load(ci, slot):
            base = e0 + ci * k
            pltpu.async_copy(second_hbm.at[pl.ds(base, k)], idx_s.at[slot], sem_g)
            pltpu.async_copy(m_hbm.at[pl.ds(base, k)], mbuf.at[slot], sem_g)

        def wait_load(slot):
            pltpu.make_async_copy(second_hbm.at[pl.ds(0, k)], idx_s.at[slot], sem_g).wait()
            pltpu.make_async_copy(m_hbm.at[pl.ds(0, k)], mbuf.at[slot], sem_g).wait()

        start_load(0, 0)

        def body(ci, carry):
            slot = lax.rem(ci, 2)

            @pl.when(ci + 1 < cpt)
            def _():
                start_load(ci + 1, 1 - slot)

            wait_load(slot)
            pltpu.sync_copy(mbuf.at[slot], acc.at[idx_s.at[slot]], add=True)
            return carry

        lax.fori_loop(0, cpt, body, None)
        plsc.subcore_barrier()

        for kk in range(5):
            r0 = sid * RPT + kk * SROWS
            pltpu.sync_copy(acc.at[pl.ds(r0, SROWS)], stage)
            pltpu.sync_copy(stage, out_hbm.at[cid, pl.ds(r0, SROWS)])

    return scatter


_gather_h = _make_gather(EH, 40)
_scatter_h = _make_scatter(EH, 40)


BN = 2000
BE = 4000


def _proj_body(h_ref, w1a_ref, w1b_ref, b1_ref, a_ref, b_ref):
    h = h_ref[...]
    a_ref[...] = jnp.dot(h, w1a_ref[...], preferred_element_type=jnp.float32) + b1_ref[...]
    b_ref[...] = jnp.dot(h, w1b_ref[...], preferred_element_type=jnp.float32)


_proj = pl.pallas_call(
    _proj_body,
    grid=(N // BN,),
    in_specs=[
        pl.BlockSpec((BN, D), lambda i: (i, 0)),
        pl.BlockSpec((D, D), lambda i: (0, 0)),
        pl.BlockSpec((D, D), lambda i: (0, 0)),
        pl.BlockSpec((1, D), lambda i: (0, 0)),
    ],
    out_specs=[
        pl.BlockSpec((BN, D), lambda i: (i, 0)),
        pl.BlockSpec((BN, D), lambda i: (i, 0)),
    ],
    out_shape=[
        jax.ShapeDtypeStruct((N, D), jnp.float32),
        jax.ShapeDtypeStruct((N, D), jnp.float32),
    ],
)


def _edge_mlp_body(c1_ref, c2_ref, w2_ref, b2_ref, m_ref):
    c = jnp.maximum(c1_ref[...] + c2_ref[...], 0.0)
    m = jnp.dot(c, w2_ref[...], preferred_element_type=jnp.float32) + b2_ref[...]
    m_ref[...] = jnp.maximum(m, 0.0)


_edge_mlp = pl.pallas_call(
    _edge_mlp_body,
    grid=(EH // BE,),
    in_specs=[
        pl.BlockSpec((BE, D), lambda i: (i, 0)),
        pl.BlockSpec((BE, D), lambda i: (i, 0)),
        pl.BlockSpec((D, D), lambda i: (0, 0)),
        pl.BlockSpec((1, D), lambda i: (0, 0)),
    ],
    out_specs=pl.BlockSpec((BE, D), lambda i: (i, 0)),
    out_shape=jax.ShapeDtypeStruct((EH, D), jnp.float32),
)


def _gru_proj_body(p0_ref, p1_ref, h_ref, gk_ref, grk_ref, gb_ref,
                   w1a_ref, w1b_ref, b1_ref, ho_ref, a_ref, b_ref):
    x = p0_ref[0] + p0_ref[1] + p1_ref[0] + p1_ref[1]
    h = h_ref[...]
    mx = jnp.dot(x, gk_ref[...], preferred_element_type=jnp.float32) + gb_ref[0:1, :]
    mh = jnp.dot(h, grk_ref[...], preferred_element_type=jnp.float32) + gb_ref[1:2, :]
    z = jax.nn.sigmoid(mx[:, :D] + mh[:, :D])
    r = jax.nn.sigmoid(mx[:, D:2 * D] + mh[:, D:2 * D])
    hh = jnp.tanh(mx[:, 2 * D:] + r * mh[:, 2 * D:])
    hn = z * h + (1.0 - z) * hh
    ho_ref[...] = hn
    a_ref[...] = jnp.dot(hn, w1a_ref[...], preferred_element_type=jnp.float32) + b1_ref[...]
    b_ref[...] = jnp.dot(hn, w1b_ref[...], preferred_element_type=jnp.float32)


_gru_proj = pl.pallas_call(
    _gru_proj_body,
    grid=(N // BN,),
    in_specs=[
        pl.BlockSpec((NC, BN, D), lambda i: (0, i, 0)),
        pl.BlockSpec((NC, BN, D), lambda i: (0, i, 0)),
        pl.BlockSpec((BN, D), lambda i: (i, 0)),
        pl.BlockSpec((D, 3 * D), lambda i: (0, 0)),
        pl.BlockSpec((D, 3 * D), lambda i: (0, 0)),
        pl.BlockSpec((2, 3 * D), lambda i: (0, 0)),
        pl.BlockSpec((D, D), lambda i: (0, 0)),
        pl.BlockSpec((D, D), lambda i: (0, 0)),
        pl.BlockSpec((1, D), lambda i: (0, 0)),
    ],
    out_specs=[
        pl.BlockSpec((BN, D), lambda i: (i, 0)),
        pl.BlockSpec((BN, D), lambda i: (i, 0)),
        pl.BlockSpec((BN, D), lambda i: (i, 0)),
    ],
    out_shape=[
        jax.ShapeDtypeStruct((N, D), jnp.float32),
        jax.ShapeDtypeStruct((N, D), jnp.float32),
        jax.ShapeDtypeStruct((N, D), jnp.float32),
    ],
)


def _readout_body(ids_ref, h_ref, wr1_ref, br1_ref, wr2_ref, br2_ref,
                  mask_ref, out_ref, acc_ref):
    i = pl.program_id(0)

    @pl.when(i == 0)
    def _zero():
        acc_ref[...] = jnp.zeros_like(acc_ref)

    ids = ids_ref[0]
    seg = lax.broadcasted_iota(jnp.int32, (G, 1), 0)
    onehot = (ids == seg).astype(jnp.float32)
    acc_ref[...] += jnp.dot(onehot, h_ref[...], preferred_element_type=jnp.float32)

    @pl.when(i == N // BN - 1)
    def _finish():
        t = jnp.dot(acc_ref[...], wr1_ref[...], preferred_element_type=jnp.float32)
        t = jnp.maximum(t + br1_ref[...], 0.0) * mask_ref[...]
        out_ref[...] = jnp.sum(t * wr2_ref[...], axis=1, keepdims=True) + br2_ref[...]


_readout = pl.pallas_call(
    _readout_body,
    grid=(N // BN,),
    in_specs=[
        pl.BlockSpec((1, 1, BN), lambda i: (i, 0, 0)),
        pl.BlockSpec((BN, D), lambda i: (i, 0)),
        pl.BlockSpec((D, D), lambda i: (0, 0)),
        pl.BlockSpec((1, D), lambda i: (0, 0)),
        pl.BlockSpec((1, D), lambda i: (0, 0)),
        pl.BlockSpec((1, 1), lambda i: (0, 0)),
        pl.BlockSpec((1, D), lambda i: (0, 0)),
    ],
    out_specs=pl.BlockSpec((G, 1), lambda i: (0, 0)),
    out_shape=jax.ShapeDtypeStruct((G, 1), jnp.float32),
    scratch_shapes=[pltpu.VMEM((G, D), jnp.float32)],
)


def kernel(link_state, states_graph_ids, states_first, states_second,
           sates_num_edges, Wm1, bm1, Wm2, bm2, gru_kernel, gru_rkernel,
           gru_bias, Wr1, br1, Wr2, br2, mask0):
    w1a = Wm1[:D]
    w1b = Wm1[D:]
    b1 = bm1.reshape(1, D)
    b2 = bm2.reshape(1, D)
    f_lo, f_hi = states_first[:EH], states_first[EH:]
    s_lo, s_hi = states_second[:EH], states_second[EH:]
    zeros_stage = jnp.zeros((SROWS, D), jnp.float32)

    h = link_state
    a, b = _proj(h, w1a, w1b, b1)
    for _ in range(T):
        c0a, c0b = _gather_h(a, b, f_lo, s_lo)
        m0 = _edge_mlp(c0a, c0b, Wm2, b2)
        c1a, c1b = _gather_h(a, b, f_hi, s_hi)
        m1 = _edge_mlp(c1a, c1b, Wm2, b2)
        p0 = _scatter_h(m0, s_lo, zeros_stage)
        p1 = _scatter_h(m1, s_hi, zeros_stage)
        h, a, b = _gru_proj(p0, p1, h, gru_kernel, gru_rkernel, gru_bias,
                            w1a, w1b, b1)

    out = _readout(states_graph_ids.reshape(N // BN, 1, BN), h, Wr1,
                   br1.reshape(1, D), Wr2.reshape(1, D), br2.reshape(1, 1),
                   mask0.reshape(1, D))
    return out


# speedup vs baseline: 1.6500x; 1.1797x over previous
import functools

import jax
import jax.numpy as jnp
from jax import lax
from jax.experimental import pallas as pl
from jax.experimental.pallas import tpu as pltpu
from jax.experimental.pallas import tpu_sc as plsc

N = 10000
E = 320000
D = 128
G = 64
T = 4

NC = 2
NS = 16
NW = NC * NS
EH = E // 2
NPAD = 10240
RPT = NPAD // NS
SROWS = RPT // 5

_mesh = plsc.VectorSubcoreMesh(core_axis_name="c", subcore_axis_name="s")


def _make_gather(ne, k):
    ept = ne // NS
    cpt = ept // k
    tch = 80
    tchn = N // tch

    @functools.partial(
        pl.kernel,
        out_type=(
            jax.ShapeDtypeStruct((ne, D), jnp.float32),
            jax.ShapeDtypeStruct((ne, D), jnp.float32),
        ),
        mesh=_mesh,
        scratch_types=[
            pltpu.VMEM((ept,), jnp.int32),
            pltpu.VMEM((2, k, D), jnp.float32),
            pltpu.VMEM((tch, D), jnp.float32),
            pltpu.VMEM_SHARED((N, D), jnp.float32),
            pltpu.SemaphoreType.DMA,
            pltpu.SemaphoreType.DMA,
        ],
    )
    def gather(a_hbm, b_hbm, first_hbm, second_hbm, c1_hbm, c2_hbm,
               idx, ring, tstage, table, sem_g, sem_w):
        cid = lax.axis_index("c")
        sid = lax.axis_index("s")

        def one_side(src_hbm, idx_hbm, c_hbm):
            for t in range(tchn // NS + 1):
                j = sid + NS * t

                @pl.when(j < tchn)
                def _():
                    pltpu.sync_copy(src_hbm.at[pl.ds(j * tch, tch)], tstage)
                    pltpu.sync_copy(tstage, table.at[pl.ds(j * tch, tch)])

            pltpu.sync_copy(idx_hbm.at[pl.ds(sid * ept, ept)], idx)
            plsc.subcore_barrier()

            def start_gather(ci, slot):
                pltpu.async_copy(table.at[idx.at[pl.ds(ci * k, k)]], ring.at[slot], sem_g)

            def wait_gather(slot):
                pltpu.make_async_copy(table.at[idx.at[pl.ds(0, k)]], ring.at[slot], sem_g).wait()

            def start_write(ci, slot):
                pltpu.async_copy(ring.at[slot], c_hbm.at[pl.ds(sid * ept + ci * k, k)], sem_w)

            def wait_write(slot):
                pltpu.make_async_copy(ring.at[slot], c_hbm.at[pl.ds(0, k)], sem_w).wait()

            start_gather(0, 0)

            def body(ci, carry):
                slot = lax.rem(ci, 2)
                nslot = 1 - slot

                @pl.when(ci + 1 < cpt)
                def _():
                    @pl.when(ci >= 1)
                    def _():
                        wait_write(nslot)
                    start_gather(ci + 1, nslot)

                wait_gather(slot)
                start_write(ci, slot)
                return carry

            lax.fori_loop(0, cpt, body, None)
            wait_write(0)
            wait_write(1)

        @pl.when(cid == 0)
        def _():
            one_side(a_hbm, first_hbm, c1_hbm)

        @pl.when(cid == 1)
        def _():
            one_side(b_hbm, second_hbm, c2_hbm)

    return gather


def _make_scatter(ne, k):
    epw = ne // NW
    cpt = epw // k

    @functools.partial(
        pl.kernel,
        out_type=jax.ShapeDtypeStruct((NC, NPAD, D), jnp.float32),
        mesh=_mesh,
        scratch_types=[
            pltpu.VMEM((2, k), jnp.int32),
            pltpu.VMEM((2, k, D), jnp.float32),
            pltpu.VMEM((SROWS, D), jnp.float32),
            pltpu.VMEM_SHARED((NPAD, D), jnp.float32),
            pltpu.SemaphoreType.DMA,
        ],
    )
    def scatter(m0_hbm, m1_hbm, second_hbm, zero_hbm, out_hbm,
                idx_s, mbuf, stage, acc, sem_g):
        cid = lax.axis_index("c")
        sid = lax.axis_index("s")
        wid = sid * NC + cid

        pltpu.sync_copy(zero_hbm, stage)
        for kk in range(5):
            pltpu.sync_copy(stage, acc.at[pl.ds(sid * RPT + kk * SROWS, SROWS)])
        plsc.subcore_barrier()

        for half, m_hbm in enumerate((m0_hbm, m1_hbm)):
            e0 = wid * epw
            s0 = half * ne + e0

            def start_load(ci, slot):
                pltpu.async_copy(second_hbm.at[pl.ds(s0 + ci * k, k)], idx_s.at[slot], sem_g)
                pltpu.async_copy(m_hbm.at[pl.ds(e0 + ci * k, k)], mbuf.at[slot], sem_g)

            def wait_load(slot):
                pltpu.make_async_copy(second_hbm.at[pl.ds(0, k)], idx_s.at[slot], sem_g).wait()
                pltpu.make_async_copy(m_hbm.at[pl.ds(0, k)], mbuf.at[slot], sem_g).wait()

            start_load(0, 0)

            def body(ci, carry):
                slot = lax.rem(ci, 2)

                @pl.when(ci + 1 < cpt)
                def _():
                    start_load(ci + 1, 1 - slot)

                wait_load(slot)
                pltpu.sync_copy(mbuf.at[slot], acc.at[idx_s.at[slot]], add=True)
                return carry

            lax.fori_loop(0, cpt, body, None)

        plsc.subcore_barrier()

        for kk in range(5):
            r0 = sid * RPT + kk * SROWS
            pltpu.sync_copy(acc.at[pl.ds(r0, SROWS)], stage)
            pltpu.sync_copy(stage, out_hbm.at[cid, pl.ds(r0, SROWS)])

    return scatter


_gather_h = _make_gather(EH, 80)
_scatter_f = _make_scatter(EH, 40)


BN = 2000
BE = 4000


def _proj_body(h_ref, w1a_ref, w1b_ref, b1_ref, a_ref, b_ref):
    h = h_ref[...]
    a_ref[...] = jnp.dot(h, w1a_ref[...], preferred_element_type=jnp.float32) + b1_ref[...]
    b_ref[...] = jnp.dot(h, w1b_ref[...], preferred_element_type=jnp.float32)


_proj = pl.pallas_call(
    _proj_body,
    grid=(N // BN,),
    in_specs=[
        pl.BlockSpec((BN, D), lambda i: (i, 0)),
        pl.BlockSpec((D, D), lambda i: (0, 0)),
        pl.BlockSpec((D, D), lambda i: (0, 0)),
        pl.BlockSpec((1, D), lambda i: (0, 0)),
    ],
    out_specs=[
        pl.BlockSpec((BN, D), lambda i: (i, 0)),
        pl.BlockSpec((BN, D), lambda i: (i, 0)),
    ],
    out_shape=[
        jax.ShapeDtypeStruct((N, D), jnp.float32),
        jax.ShapeDtypeStruct((N, D), jnp.float32),
    ],
)


def _edge_mlp_body(c1_ref, c2_ref, w2_ref, b2_ref, m_ref):
    c = jnp.maximum(c1_ref[...] + c2_ref[...], 0.0)
    m = jnp.dot(c, w2_ref[...], preferred_element_type=jnp.float32) + b2_ref[...]
    m_ref[...] = jnp.maximum(m, 0.0)


_edge_mlp = pl.pallas_call(
    _edge_mlp_body,
    grid=(EH // BE,),
    in_specs=[
        pl.BlockSpec((BE, D), lambda i: (i, 0)),
        pl.BlockSpec((BE, D), lambda i: (i, 0)),
        pl.BlockSpec((D, D), lambda i: (0, 0)),
        pl.BlockSpec((1, D), lambda i: (0, 0)),
    ],
    out_specs=pl.BlockSpec((BE, D), lambda i: (i, 0)),
    out_shape=jax.ShapeDtypeStruct((EH, D), jnp.float32),
)


def _gru_proj_body(p_ref, h_ref, gk_ref, grk_ref, gb_ref,
                   w1a_ref, w1b_ref, b1_ref, ho_ref, a_ref, b_ref):
    x = p_ref[0] + p_ref[1]
    h = h_ref[...]
    mx = jnp.dot(x, gk_ref[...], preferred_element_type=jnp.float32) + gb_ref[0:1, :]
    mh = jnp.dot(h, grk_ref[...], preferred_element_type=jnp.float32) + gb_ref[1:2, :]
    z = jax.nn.sigmoid(mx[:, :D] + mh[:, :D])
    r = jax.nn.sigmoid(mx[:, D:2 * D] + mh[:, D:2 * D])
    hh = jnp.tanh(mx[:, 2 * D:] + r * mh[:, 2 * D:])
    hn = z * h + (1.0 - z) * hh
    ho_ref[...] = hn
    a_ref[...] = jnp.dot(hn, w1a_ref[...], preferred_element_type=jnp.float32) + b1_ref[...]
    b_ref[...] = jnp.dot(hn, w1b_ref[...], preferred_element_type=jnp.float32)


_gru_proj = pl.pallas_call(
    _gru_proj_body,
    grid=(N // BN,),
    in_specs=[
        pl.BlockSpec((NC, BN, D), lambda i: (0, i, 0)),
        pl.BlockSpec((BN, D), lambda i: (i, 0)),
        pl.BlockSpec((D, 3 * D), lambda i: (0, 0)),
        pl.BlockSpec((D, 3 * D), lambda i: (0, 0)),
        pl.BlockSpec((2, 3 * D), lambda i: (0, 0)),
        pl.BlockSpec((D, D), lambda i: (0, 0)),
        pl.BlockSpec((D, D), lambda i: (0, 0)),
        pl.BlockSpec((1, D), lambda i: (0, 0)),
    ],
    out_specs=[
        pl.BlockSpec((BN, D), lambda i: (i, 0)),
        pl.BlockSpec((BN, D), lambda i: (i, 0)),
        pl.BlockSpec((BN, D), lambda i: (i, 0)),
    ],
    out_shape=[
        jax.ShapeDtypeStruct((N, D), jnp.float32),
        jax.ShapeDtypeStruct((N, D), jnp.float32),
        jax.ShapeDtypeStruct((N, D), jnp.float32),
    ],
)


def _readout_body(ids_ref, h_ref, wr1_ref, br1_ref, wr2_ref, br2_ref,
                  mask_ref, out_ref, acc_ref):
    i = pl.program_id(0)

    @pl.when(i == 0)
    def _zero():
        acc_ref[...] = jnp.zeros_like(acc_ref)

    ids = ids_ref[0]
    seg = lax.broadcasted_iota(jnp.int32, (G, 1), 0)
    onehot = (ids == seg).astype(jnp.float32)
    acc_ref[...] += jnp.dot(onehot, h_ref[...], preferred_element_type=jnp.float32)

    @pl.when(i == N // BN - 1)
    def _finish():
        t = jnp.dot(acc_ref[...], wr1_ref[...], preferred_element_type=jnp.float32)
        t = jnp.maximum(t + br1_ref[...], 0.0) * mask_ref[...]
        out_ref[...] = jnp.sum(t * wr2_ref[...], axis=1, keepdims=True) + br2_ref[...]


_readout = pl.pallas_call(
    _readout_body,
    grid=(N // BN,),
    in_specs=[
        pl.BlockSpec((1, 1, BN), lambda i: (i, 0, 0)),
        pl.BlockSpec((BN, D), lambda i: (i, 0)),
        pl.BlockSpec((D, D), lambda i: (0, 0)),
        pl.BlockSpec((1, D), lambda i: (0, 0)),
        pl.BlockSpec((1, D), lambda i: (0, 0)),
        pl.BlockSpec((1, 1), lambda i: (0, 0)),
        pl.BlockSpec((1, D), lambda i: (0, 0)),
    ],
    out_specs=pl.BlockSpec((G, 1), lambda i: (0, 0)),
    out_shape=jax.ShapeDtypeStruct((G, 1), jnp.float32),
    scratch_shapes=[pltpu.VMEM((G, D), jnp.float32)],
)


def kernel(link_state, states_graph_ids, states_first, states_second,
           sates_num_edges, Wm1, bm1, Wm2, bm2, gru_kernel, gru_rkernel,
           gru_bias, Wr1, br1, Wr2, br2, mask0):
    w1a = Wm1[:D]
    w1b = Wm1[D:]
    b1 = bm1.reshape(1, D)
    b2 = bm2.reshape(1, D)
    f_lo, f_hi = states_first[:EH], states_first[EH:]
    s_lo, s_hi = states_second[:EH], states_second[EH:]
    zeros_stage = jnp.zeros((SROWS, D), jnp.float32)

    h = link_state
    a, b = _proj(h, w1a, w1b, b1)
    for _ in range(T):
        c0a, c0b = _gather_h(a, b, f_lo, s_lo)
        m0 = _edge_mlp(c0a, c0b, Wm2, b2)
        c1a, c1b = _gather_h(a, b, f_hi, s_hi)
        m1 = _edge_mlp(c1a, c1b, Wm2, b2)
        p = _scatter_f(m0, m1, states_second, zeros_stage)
        h, a, b = _gru_proj(p, h, gru_kernel, gru_rkernel, gru_bias,
                            w1a, w1b, b1)

    out = _readout(states_graph_ids.reshape(N // BN, 1, BN), h, Wr1,
                   br1.reshape(1, D), Wr2.reshape(1, D), br2.reshape(1, 1),
                   mask0.reshape(1, D))
    return out


# speedup vs baseline: 1.9106x; 1.1579x over previous
import functools

import jax
import jax.numpy as jnp
from jax import lax
from jax.experimental import pallas as pl
from jax.experimental.pallas import tpu as pltpu
from jax.experimental.pallas import tpu_sc as plsc

N = 10000
E = 320000
D = 128
G = 64
T = 4

NC = 2
NS = 16
NW = NC * NS
EH = E // 2
NPAD = 10240
RPT = NPAD // NS
SROWS = RPT // 5

_mesh = plsc.VectorSubcoreMesh(core_axis_name="c", subcore_axis_name="s")


def _make_gather(ne):
    ept = ne // NS
    k = 128
    cpt = ept // k
    tail = ept - cpt * k
    tcn = N // k
    ttail = N - tcn * k

    @functools.partial(
        pl.kernel,
        out_type=(
            jax.ShapeDtypeStruct((ne, D), jnp.float32),
            jax.ShapeDtypeStruct((ne, D), jnp.float32),
        ),
        mesh=_mesh,
        scratch_types=[
            pltpu.VMEM((ept,), jnp.int32),
            pltpu.VMEM((2, k, D), jnp.float32),
            pltpu.VMEM_SHARED((N, D), jnp.float32),
            pltpu.SemaphoreType.DMA,
            pltpu.SemaphoreType.DMA,
        ],
    )
    def gather(a_hbm, b_hbm, first_hbm, second_hbm, c1_hbm, c2_hbm,
               idx, ring, table, sem_g, sem_w):
        cid = lax.axis_index("c")
        sid = lax.axis_index("s")

        def one_side(src_hbm, idx_hbm, c_hbm):
            for t in range(tcn // NS + 1):
                j = sid + NS * t

                @pl.when(j < tcn)
                def _():
                    pltpu.sync_copy(src_hbm.at[pl.ds(j * k, k)], ring.at[0])
                    pltpu.sync_copy(ring.at[0], table.at[pl.ds(j * k, k)])

            @pl.when(sid == 0)
            def _():
                pltpu.sync_copy(src_hbm.at[pl.ds(tcn * k, ttail)],
                                ring.at[0, pl.ds(0, ttail)])
                pltpu.sync_copy(ring.at[0, pl.ds(0, ttail)],
                                table.at[pl.ds(tcn * k, ttail)])

            pltpu.sync_copy(idx_hbm.at[pl.ds(sid * ept, ept)], idx)
            plsc.subcore_barrier()

            def start_gather(ci, slot):
                pltpu.async_copy(table.at[idx.at[pl.ds(ci * k, k)]], ring.at[slot], sem_g)

            def wait_gather(slot):
                pltpu.make_async_copy(table.at[idx.at[pl.ds(0, k)]], ring.at[slot], sem_g).wait()

            def start_write(ci, slot):
                pltpu.async_copy(ring.at[slot], c_hbm.at[pl.ds(sid * ept + ci * k, k)], sem_w)

            def wait_write(slot):
                pltpu.make_async_copy(ring.at[slot], c_hbm.at[pl.ds(0, k)], sem_w).wait()

            start_gather(0, 0)

            def body(ci, carry):
                slot = lax.rem(ci, 2)
                nslot = 1 - slot

                @pl.when(ci + 1 < cpt)
                def _():
                    @pl.when(ci >= 1)
                    def _():
                        wait_write(nslot)
                    start_gather(ci + 1, nslot)

                wait_gather(slot)
                start_write(ci, slot)
                return carry

            lax.fori_loop(0, cpt, body, None)
            wait_write(0)
            wait_write(1)

            pltpu.async_copy(table.at[idx.at[pl.ds(cpt * k, tail)]],
                             ring.at[0, pl.ds(0, tail)], sem_g).wait()
            pltpu.sync_copy(ring.at[0, pl.ds(0, tail)],
                            c_hbm.at[pl.ds(sid * ept + cpt * k, tail)])

        @pl.when(cid == 0)
        def _():
            one_side(a_hbm, first_hbm, c1_hbm)

        @pl.when(cid == 1)
        def _():
            one_side(b_hbm, second_hbm, c2_hbm)

    return gather


def _make_scatter(ne):
    epw = ne // NW
    k = 104
    cpt = epw // k
    tail = epw - cpt * k

    @functools.partial(
        pl.kernel,
        out_type=jax.ShapeDtypeStruct((NC, NPAD, D), jnp.float32),
        mesh=_mesh,
        scratch_types=[
            pltpu.VMEM((2, k), jnp.int32),
            pltpu.VMEM((2, k, D), jnp.float32),
            pltpu.VMEM((tail,), jnp.int32),
            pltpu.VMEM((tail, D), jnp.float32),
            pltpu.VMEM((SROWS, D), jnp.float32),
            pltpu.VMEM_SHARED((NPAD, D), jnp.float32),
            pltpu.SemaphoreType.DMA,
        ],
    )
    def scatter(m0_hbm, m1_hbm, second_hbm, zero_hbm, out_hbm,
                idx_s, mbuf, idx_t, mbuf_t, stage, acc, sem_g):
        cid = lax.axis_index("c")
        sid = lax.axis_index("s")
        wid = sid * NC + cid

        pltpu.sync_copy(zero_hbm, stage)
        for kk in range(5):
            pltpu.sync_copy(stage, acc.at[pl.ds(sid * RPT + kk * SROWS, SROWS)])
        plsc.subcore_barrier()

        for half, m_hbm in enumerate((m0_hbm, m1_hbm)):
            e0 = wid * epw
            s0 = half * ne + e0

            def start_load(ci, slot):
                pltpu.async_copy(second_hbm.at[pl.ds(s0 + ci * k, k)], idx_s.at[slot], sem_g)
                pltpu.async_copy(m_hbm.at[pl.ds(e0 + ci * k, k)], mbuf.at[slot], sem_g)

            def wait_load(slot):
                pltpu.make_async_copy(second_hbm.at[pl.ds(0, k)], idx_s.at[slot], sem_g).wait()
                pltpu.make_async_copy(m_hbm.at[pl.ds(0, k)], mbuf.at[slot], sem_g).wait()

            start_load(0, 0)

            def body(ci, carry):
                slot = lax.rem(ci, 2)

                @pl.when(ci + 1 < cpt)
                def _():
                    start_load(ci + 1, 1 - slot)

                wait_load(slot)
                pltpu.sync_copy(mbuf.at[slot], acc.at[idx_s.at[slot]], add=True)
                return carry

            lax.fori_loop(0, cpt, body, None)

            pltpu.sync_copy(second_hbm.at[pl.ds(s0 + cpt * k, tail)], idx_t)
            pltpu.sync_copy(m_hbm.at[pl.ds(e0 + cpt * k, tail)], mbuf_t)
            pltpu.sync_copy(mbuf_t, acc.at[idx_t], add=True)

        plsc.subcore_barrier()

        for kk in range(5):
            r0 = sid * RPT + kk * SROWS
            pltpu.sync_copy(acc.at[pl.ds(r0, SROWS)], stage)
            pltpu.sync_copy(stage, out_hbm.at[cid, pl.ds(r0, SROWS)])

    return scatter


_gather_h = _make_gather(EH)
_scatter_f = _make_scatter(EH)


BN = 2000
BE = 4000


def _proj_body(h_ref, w1a_ref, w1b_ref, b1_ref, a_ref, b_ref):
    h = h_ref[...]
    a_ref[...] = jnp.dot(h, w1a_ref[...], preferred_element_type=jnp.float32) + b1_ref[...]
    b_ref[...] = jnp.dot(h, w1b_ref[...], preferred_element_type=jnp.float32)


_proj = pl.pallas_call(
    _proj_body,
    grid=(N // BN,),
    in_specs=[
        pl.BlockSpec((BN, D), lambda i: (i, 0)),
        pl.BlockSpec((D, D), lambda i: (0, 0)),
        pl.BlockSpec((D, D), lambda i: (0, 0)),
        pl.BlockSpec((1, D), lambda i: (0, 0)),
    ],
    out_specs=[
        pl.BlockSpec((BN, D), lambda i: (i, 0)),
        pl.BlockSpec((BN, D), lambda i: (i, 0)),
    ],
    out_shape=[
        jax.ShapeDtypeStruct((N, D), jnp.float32),
        jax.ShapeDtypeStruct((N, D), jnp.float32),
    ],
)


def _edge_mlp_body(c1_ref, c2_ref, w2_ref, b2_ref, m_ref):
    c = jnp.maximum(c1_ref[...] + c2_ref[...], 0.0)
    m = jnp.dot(c, w2_ref[...], preferred_element_type=jnp.float32) + b2_ref[...]
    m_ref[...] = jnp.maximum(m, 0.0)


_edge_mlp = pl.pallas_call(
    _edge_mlp_body,
    grid=(EH // BE,),
    in_specs=[
        pl.BlockSpec((BE, D), lambda i: (i, 0)),
        pl.BlockSpec((BE, D), lambda i: (i, 0)),
        pl.BlockSpec((D, D), lambda i: (0, 0)),
        pl.BlockSpec((1, D), lambda i: (0, 0)),
    ],
    out_specs=pl.BlockSpec((BE, D), lambda i: (i, 0)),
    out_shape=jax.ShapeDtypeStruct((EH, D), jnp.float32),
)


def _gru_proj_body(p_ref, h_ref, gk_ref, grk_ref, gb_ref,
                   w1a_ref, w1b_ref, b1_ref, ho_ref, a_ref, b_ref):
    x = p_ref[0] + p_ref[1]
    h = h_ref[...]
    mx = jnp.dot(x, gk_ref[...], preferred_element_type=jnp.float32) + gb_ref[0:1, :]
    mh = jnp.dot(h, grk_ref[...], preferred_element_type=jnp.float32) + gb_ref[1:2, :]
    z = jax.nn.sigmoid(mx[:, :D] + mh[:, :D])
    r = jax.nn.sigmoid(mx[:, D:2 * D] + mh[:, D:2 * D])
    hh = jnp.tanh(mx[:, 2 * D:] + r * mh[:, 2 * D:])
    hn = z * h + (1.0 - z) * hh
    ho_ref[...] = hn
    a_ref[...] = jnp.dot(hn, w1a_ref[...], preferred_element_type=jnp.float32) + b1_ref[...]
    b_ref[...] = jnp.dot(hn, w1b_ref[...], preferred_element_type=jnp.float32)


_gru_proj = pl.pallas_call(
    _gru_proj_body,
    grid=(N // BN,),
    in_specs=[
        pl.BlockSpec((NC, BN, D), lambda i: (0, i, 0)),
        pl.BlockSpec((BN, D), lambda i: (i, 0)),
        pl.BlockSpec((D, 3 * D), lambda i: (0, 0)),
        pl.BlockSpec((D, 3 * D), lambda i: (0, 0)),
        pl.BlockSpec((2, 3 * D), lambda i: (0, 0)),
        pl.BlockSpec((D, D), lambda i: (0, 0)),
        pl.BlockSpec((D, D), lambda i: (0, 0)),
        pl.BlockSpec((1, D), lambda i: (0, 0)),
    ],
    out_specs=[
        pl.BlockSpec((BN, D), lambda i: (i, 0)),
        pl.BlockSpec((BN, D), lambda i: (i, 0)),
        pl.BlockSpec((BN, D), lambda i: (i, 0)),
    ],
    out_shape=[
        jax.ShapeDtypeStruct((N, D), jnp.float32),
        jax.ShapeDtypeStruct((N, D), jnp.float32),
        jax.ShapeDtypeStruct((N, D), jnp.float32),
    ],
)


def _readout_body(ids_ref, h_ref, wr1_ref, br1_ref, wr2_ref, br2_ref,
                  mask_ref, out_ref, acc_ref):
    i = pl.program_id(0)

    @pl.when(i == 0)
    def _zero():
        acc_ref[...] = jnp.zeros_like(acc_ref)

    ids = ids_ref[0]
    seg = lax.broadcasted_iota(jnp.int32, (G, 1), 0)
    onehot = (ids == seg).astype(jnp.float32)
    acc_ref[...] += jnp.dot(onehot, h_ref[...], preferred_element_type=jnp.float32)

    @pl.when(i == N // BN - 1)
    def _finish():
        t = jnp.dot(acc_ref[...], wr1_ref[...], preferred_element_type=jnp.float32)
        t = jnp.maximum(t + br1_ref[...], 0.0) * mask_ref[...]
        out_ref[...] = jnp.sum(t * wr2_ref[...], axis=1, keepdims=True) + br2_ref[...]


_readout = pl.pallas_call(
    _readout_body,
    grid=(N // BN,),
    in_specs=[
        pl.BlockSpec((1, 1, BN), lambda i: (i, 0, 0)),
        pl.BlockSpec((BN, D), lambda i: (i, 0)),
        pl.BlockSpec((D, D), lambda i: (0, 0)),
        pl.BlockSpec((1, D), lambda i: (0, 0)),
        pl.BlockSpec((1, D), lambda i: (0, 0)),
        pl.BlockSpec((1, 1), lambda i: (0, 0)),
        pl.BlockSpec((1, D), lambda i: (0, 0)),
    ],
    out_specs=pl.BlockSpec((G, 1), lambda i: (0, 0)),
    out_shape=jax.ShapeDtypeStruct((G, 1), jnp.float32),
    scratch_shapes=[pltpu.VMEM((G, D), jnp.float32)],
)


def kernel(link_state, states_graph_ids, states_first, states_second,
           sates_num_edges, Wm1, bm1, Wm2, bm2, gru_kernel, gru_rkernel,
           gru_bias, Wr1, br1, Wr2, br2, mask0):
    w1a = Wm1[:D]
    w1b = Wm1[D:]
    b1 = bm1.reshape(1, D)
    b2 = bm2.reshape(1, D)
    f_lo, f_hi = states_first[:EH], states_first[EH:]
    s_lo, s_hi = states_second[:EH], states_second[EH:]
    zeros_stage = jnp.zeros((SROWS, D), jnp.float32)

    h = link_state
    a, b = _proj(h, w1a, w1b, b1)
    for _ in range(T):
        c0a, c0b = _gather_h(a, b, f_lo, s_lo)
        m0 = _edge_mlp(c0a, c0b, Wm2, b2)
        c1a, c1b = _gather_h(a, b, f_hi, s_hi)
        m1 = _edge_mlp(c1a, c1b, Wm2, b2)
        p = _scatter_f(m0, m1, states_second, zeros_stage)
        h, a, b = _gru_proj(p, h, gru_kernel, gru_rkernel, gru_bias,
                            w1a, w1b, b1)

    out = _readout(states_graph_ids.reshape(N // BN, 1, BN), h, Wr1,
                   br1.reshape(1, D), Wr2.reshape(1, D), br2.reshape(1, 1),
                   mask0.reshape(1, D))
    return out
